# Initial kernel scaffold; baseline (speedup 1.0000x reference)
import jax
import jax.numpy as jnp
from jax.experimental import pallas as pl

_R = 64


def _body(xr_ref, w1a_ref, b1a_ref, w2a_ref, b2a_ref,
          w1b_ref, b1b_ref, w2b_ref, b2b_ref,
          ws1_ref, bs1_ref, ws3_ref, bs3_ref, ws2_ref, bs2_ref,
          out_ref):
    f32 = jnp.float32
    x = xr_ref[0]
    h1 = jnp.dot(x, ws1_ref[...], preferred_element_type=f32) + bs1_ref[...]
    h3 = jnp.dot(x, ws3_ref[...], preferred_element_type=f32) + bs3_ref[...]
    shared = jnp.dot(jax.nn.silu(h1) * h3, ws2_ref[...],
                     preferred_element_type=f32) + bs2_ref[...]
    ha = jax.nn.gelu(jnp.dot(x, w1a_ref[0], preferred_element_type=f32)
                     + b1a_ref[0], approximate=False)
    ea = jnp.dot(ha, w2a_ref[0], preferred_element_type=f32) + b2a_ref[0]
    hb = jax.nn.gelu(jnp.dot(x, w1b_ref[0], preferred_element_type=f32)
                     + b1b_ref[0], approximate=False)
    eb = jnp.dot(hb, w2b_ref[0], preferred_element_type=f32) + b2b_ref[0]
    out_ref[0] = 0.5 * shared + 0.25 * (ea + eb)


def kernel(x, t_emb, Ws1, bs1, Ws3, bs3, Ws2, bs2, W1, b1, W2, b2):
    B, T, C = x.shape
    N = B * T
    J = N // _R
    E, _, HR = W1.shape
    HS = Ws1.shape[1]
    f32 = jnp.float32

    xr = x.reshape(J, _R, C).transpose(1, 0, 2)
    b1r = b1[:, None, :]
    b2r = b2[:, None, :]
    bs1r = bs1[None, :]
    bs3r = bs3[None, :]
    bs2r = bs2[None, :]

    out = pl.pallas_call(
        _body,
        grid=(_R,),
        in_specs=[
            pl.BlockSpec((1, J, C), lambda r: (r, 0, 0)),
            pl.BlockSpec((1, C, HR), lambda r: ((3 * r) % _R, 0, 0)),
            pl.BlockSpec((1, 1, HR), lambda r: ((3 * r) % _R, 0, 0)),
            pl.BlockSpec((1, HR, C), lambda r: ((3 * r) % _R, 0, 0)),
            pl.BlockSpec((1, 1, C), lambda r: ((3 * r) % _R, 0, 0)),
            pl.BlockSpec((1, C, HR), lambda r: ((3 * r + 47) % _R, 0, 0)),
            pl.BlockSpec((1, 1, HR), lambda r: ((3 * r + 47) % _R, 0, 0)),
            pl.BlockSpec((1, HR, C), lambda r: ((3 * r + 47) % _R, 0, 0)),
            pl.BlockSpec((1, 1, C), lambda r: ((3 * r + 47) % _R, 0, 0)),
            pl.BlockSpec((C, HS), lambda r: (0, 0)),
            pl.BlockSpec((1, HS), lambda r: (0, 0)),
            pl.BlockSpec((C, HS), lambda r: (0, 0)),
            pl.BlockSpec((1, HS), lambda r: (0, 0)),
            pl.BlockSpec((HS, C), lambda r: (0, 0)),
            pl.BlockSpec((1, C), lambda r: (0, 0)),
        ],
        out_specs=pl.BlockSpec((1, J, C), lambda r: (r, 0, 0)),
        out_shape=jax.ShapeDtypeStruct((_R, J, C), f32),
    )(xr, W1, b1r, W2, b2r, W1, b1r, W2, b2r,
      Ws1, bs1r, Ws3, bs3r, Ws2, bs2r)

    return out.transpose(1, 0, 2).reshape(B, T, C)


# speedup vs baseline: 2.2020x; 2.2020x over previous
import jax
import jax.numpy as jnp
from jax.experimental import pallas as pl

_R = 64


def _gelu_exact(v):
    return 0.5 * v * (1.0 + jax.lax.erf(v * 0.7071067811865476))


def _body(xr_ref, w1a_ref, b1a_ref, w2a_ref, b2a_ref,
          w1b_ref, b1b_ref, w2b_ref, b2b_ref,
          ws1_ref, bs1_ref, ws3_ref, bs3_ref, ws2_ref, bs2_ref,
          out_ref):
    f32 = jnp.float32
    x = xr_ref[0]
    h1 = jnp.dot(x, ws1_ref[...], preferred_element_type=f32) + bs1_ref[...]
    h3 = jnp.dot(x, ws3_ref[...], preferred_element_type=f32) + bs3_ref[...]
    shared = jnp.dot(jax.nn.silu(h1) * h3, ws2_ref[...],
                     preferred_element_type=f32) + bs2_ref[...]
    ha = _gelu_exact(jnp.dot(x, w1a_ref[0], preferred_element_type=f32)
                     + b1a_ref[0])
    ea = jnp.dot(ha, w2a_ref[0], preferred_element_type=f32) + b2a_ref[0]
    hb = _gelu_exact(jnp.dot(x, w1b_ref[0], preferred_element_type=f32)
                     + b1b_ref[0])
    eb = jnp.dot(hb, w2b_ref[0], preferred_element_type=f32) + b2b_ref[0]
    out_ref[0] = 0.5 * shared + 0.25 * (ea + eb)


def kernel(x, t_emb, Ws1, bs1, Ws3, bs3, Ws2, bs2, W1, b1, W2, b2):
    B, T, C = x.shape
    N = B * T
    J = N // _R
    E, _, HR = W1.shape
    HS = Ws1.shape[1]
    f32 = jnp.float32

    xr = x.reshape(J, _R, C).transpose(1, 0, 2)
    b1r = b1[:, None, :]
    b2r = b2[:, None, :]
    bs1r = bs1[None, :]
    bs3r = bs3[None, :]
    bs2r = bs2[None, :]

    out = pl.pallas_call(
        _body,
        grid=(_R,),
        in_specs=[
            pl.BlockSpec((1, J, C), lambda r: (r, 0, 0)),
            pl.BlockSpec((1, C, HR), lambda r: ((3 * r) % _R, 0, 0)),
            pl.BlockSpec((1, 1, HR), lambda r: ((3 * r) % _R, 0, 0)),
            pl.BlockSpec((1, HR, C), lambda r: ((3 * r) % _R, 0, 0)),
            pl.BlockSpec((1, 1, C), lambda r: ((3 * r) % _R, 0, 0)),
            pl.BlockSpec((1, C, HR), lambda r: ((3 * r + 47) % _R, 0, 0)),
            pl.BlockSpec((1, 1, HR), lambda r: ((3 * r + 47) % _R, 0, 0)),
            pl.BlockSpec((1, HR, C), lambda r: ((3 * r + 47) % _R, 0, 0)),
            pl.BlockSpec((1, 1, C), lambda r: ((3 * r + 47) % _R, 0, 0)),
            pl.BlockSpec((C, HS), lambda r: (0, 0)),
            pl.BlockSpec((1, HS), lambda r: (0, 0)),
            pl.BlockSpec((C, HS), lambda r: (0, 0)),
            pl.BlockSpec((1, HS), lambda r: (0, 0)),
            pl.BlockSpec((HS, C), lambda r: (0, 0)),
            pl.BlockSpec((1, C), lambda r: (0, 0)),
        ],
        out_specs=pl.BlockSpec((1, J, C), lambda r: (r, 0, 0)),
        out_shape=jax.ShapeDtypeStruct((_R, J, C), f32),
    )(xr, W1, b1r, W2, b2r, W1, b1r, W2, b2r,
      Ws1, bs1r, Ws3, bs3r, Ws2, bs2r)

    return out.transpose(1, 0, 2).reshape(B, T, C)
